# Initial kernel scaffold; baseline (speedup 1.0000x reference)
import jax
import jax.numpy as jnp
from jax.experimental import pallas as pl


def kernel(x, idx, W1p, b1p, W2p, b2p, W1r, b1r, W2r, b2r):
    raise NotImplementedError("write your pallas kernel here")


# speedup vs baseline: 2.7967x; 2.7967x over previous
import functools

import jax
import jax.numpy as jnp
from jax import lax
from jax.experimental import pallas as pl
from jax.experimental.pallas import tpu as pltpu
from jax.experimental.pallas import tpu_sc as plsc

N = 320000
D = 128
H = 256
S = 10000

_NC = 2
_NS = 16
_NW = _NC * _NS
_RB = 128
_NBLK = N // _RB
_SP = 10240
_SLAB = _SP // _NS


def _mlp_body(x_ref, w1_ref, b1_ref, w2_ref, b2_ref, out_ref):
    h = jnp.dot(x_ref[...], w1_ref[...], preferred_element_type=jnp.float32)
    h = jnp.maximum(h + b1_ref[...], 0.0)
    o = jnp.dot(h, w2_ref[...], preferred_element_type=jnp.float32)
    out_ref[...] = o + b2_ref[...]


def _combine_mlp_body(p_ref, w1_ref, b1_ref, w2_ref, b2_ref, out_ref):
    seg = p_ref[0] + p_ref[1]
    h = jnp.dot(seg, w1_ref[...], preferred_element_type=jnp.float32)
    h = jnp.maximum(h + b1_ref[...], 0.0)
    o = jnp.dot(h, w2_ref[...], preferred_element_type=jnp.float32)
    out_ref[...] = o + b2_ref[...]


def _point_net(x, w1, b1, w2, b2, block_rows=2560):
    grid = (x.shape[0] // block_rows,)
    return pl.pallas_call(
        _mlp_body,
        grid=grid,
        in_specs=[
            pl.BlockSpec((block_rows, D), lambda i: (i, 0)),
            pl.BlockSpec((D, H), lambda i: (0, 0)),
            pl.BlockSpec((H,), lambda i: (0,)),
            pl.BlockSpec((H, D), lambda i: (0, 0)),
            pl.BlockSpec((D,), lambda i: (0,)),
        ],
        out_specs=pl.BlockSpec((block_rows, D), lambda i: (i, 0)),
        out_shape=jax.ShapeDtypeStruct((x.shape[0], D), jnp.float32),
    )(x, w1, b1, w2, b2)


def _reduce_net(parts, w1, b1, w2, b2, block_rows=2000):
    grid = (S // block_rows,)
    return pl.pallas_call(
        _combine_mlp_body,
        grid=grid,
        in_specs=[
            pl.BlockSpec((2, block_rows, D), lambda i: (0, i, 0)),
            pl.BlockSpec((D, H), lambda i: (0, 0)),
            pl.BlockSpec((H,), lambda i: (0,)),
            pl.BlockSpec((H, D), lambda i: (0, 0)),
            pl.BlockSpec((D,), lambda i: (0,)),
        ],
        out_specs=pl.BlockSpec((block_rows, D), lambda i: (i, 0)),
        out_shape=jax.ShapeDtypeStruct((S, D), jnp.float32),
    )(parts, w1, b1, w2, b2)


def _segment_sum_sc_body(pt_hbm, idx_hbm, zeros_hbm, out_hbm,
                         idx_v, rows_v, acc_sh):
    c = lax.axis_index("c")
    s = lax.axis_index("s")
    wid = c * _NS + s

    pltpu.sync_copy(zeros_hbm, acc_sh.at[pl.ds(s * _SLAB, _SLAB)])
    plsc.subcore_barrier()

    nb = 78 + jnp.where(wid < _NBLK - 78 * _NW, 1, 0)

    def body(k, carry):
        b = k * _NW + wid
        pltpu.sync_copy(idx_hbm.at[b], idx_v)
        pltpu.sync_copy(pt_hbm.at[pl.ds(b * _RB, _RB)], rows_v)
        pltpu.sync_copy(rows_v, acc_sh.at[idx_v], add=True)
        return carry

    lax.fori_loop(0, nb, body, 0)
    plsc.subcore_barrier()

    pltpu.sync_copy(acc_sh.at[pl.ds(s * _SLAB, _SLAB)],
                    out_hbm.at[c, pl.ds(s * _SLAB, _SLAB)])


def _segment_sum_sc(pt, idx2d, zeros_slab):
    mesh = plsc.VectorSubcoreMesh(core_axis_name="c", subcore_axis_name="s")
    k = pl.kernel(
        _segment_sum_sc_body,
        out_type=jax.ShapeDtypeStruct((_NC, _SP, D), jnp.float32),
        mesh=mesh,
        scratch_types=[
            pltpu.VMEM((_RB,), jnp.int32),
            pltpu.VMEM((_RB, D), jnp.float32),
            pltpu.VMEM_SHARED((_SP, D), jnp.float32),
        ],
    )
    return k(pt, idx2d, zeros_slab)


def kernel(x, idx, W1p, b1p, W2p, b2p, W1r, b1r, W2r, b2r):
    pt = _point_net(x, W1p, b1p, W2p, b2p)
    idx2d = idx.astype(jnp.int32).reshape(_NBLK, _RB)
    zeros_slab = jnp.zeros((_SLAB, D), jnp.float32)
    parts = _segment_sum_sc(pt, idx2d, zeros_slab)
    return _reduce_net(parts, W1r, b1r, W2r, b2r)


# speedup vs baseline: 3.2744x; 1.1708x over previous
import functools

import jax
import jax.numpy as jnp
from jax import lax
from jax.experimental import pallas as pl
from jax.experimental.pallas import tpu as pltpu
from jax.experimental.pallas import tpu_sc as plsc

N = 320000
D = 128
H = 256
S = 10000

_NC = 2
_NS = 16
_NW = _NC * _NS
_RB = 128
_NBLK = N // _RB
_SP = 10240
_SLAB = _SP // _NS
_WBLK = 80
_NBLK_PAD = _WBLK * _NW
_CB = 1


def _mlp_body(x_ref, w1_ref, b1_ref, w2_ref, b2_ref, out_ref):
    h = jnp.dot(x_ref[...], w1_ref[...], preferred_element_type=jnp.float32)
    h = jnp.maximum(h + b1_ref[...], 0.0)
    o = jnp.dot(h, w2_ref[...], preferred_element_type=jnp.float32)
    out_ref[...] = o + b2_ref[...]


def _combine_mlp_body(p_ref, w1_ref, b1_ref, w2_ref, b2_ref, out_ref):
    seg = p_ref[0] + p_ref[1]
    h = jnp.dot(seg, w1_ref[...], preferred_element_type=jnp.float32)
    h = jnp.maximum(h + b1_ref[...], 0.0)
    o = jnp.dot(h, w2_ref[...], preferred_element_type=jnp.float32)
    out_ref[...] = o + b2_ref[...]


def _point_net(x, w1, b1, w2, b2, block_rows=2560):
    grid = (x.shape[0] // block_rows,)
    return pl.pallas_call(
        _mlp_body,
        grid=grid,
        in_specs=[
            pl.BlockSpec((block_rows, D), lambda i: (i, 0)),
            pl.BlockSpec((D, H), lambda i: (0, 0)),
            pl.BlockSpec((H,), lambda i: (0,)),
            pl.BlockSpec((H, D), lambda i: (0, 0)),
            pl.BlockSpec((D,), lambda i: (0,)),
        ],
        out_specs=pl.BlockSpec((block_rows, D), lambda i: (i, 0)),
        out_shape=jax.ShapeDtypeStruct((x.shape[0], D), jnp.float32),
    )(x, w1, b1, w2, b2)


def _reduce_net(parts, w1, b1, w2, b2, block_rows=2000):
    grid = (S // block_rows,)
    return pl.pallas_call(
        _combine_mlp_body,
        grid=grid,
        in_specs=[
            pl.BlockSpec((2, block_rows, D), lambda i: (0, i, 0)),
            pl.BlockSpec((D, H), lambda i: (0, 0)),
            pl.BlockSpec((H,), lambda i: (0,)),
            pl.BlockSpec((H, D), lambda i: (0, 0)),
            pl.BlockSpec((D,), lambda i: (0,)),
        ],
        out_specs=pl.BlockSpec((block_rows, D), lambda i: (i, 0)),
        out_shape=jax.ShapeDtypeStruct((S, D), jnp.float32),
    )(parts, w1, b1, w2, b2)


def _segment_sum_sc_body(pt_hbm, idx_hbm, zeros_hbm, out_hbm,
                         idx_v, buf0_v, buf1_v, acc_sh,
                         sem_l0, sem_l1, sem_s0, sem_s1):
    c = lax.axis_index("c")
    s = lax.axis_index("s")
    wid = c * _NS + s
    start = wid * _WBLK
    npair = jnp.where(wid == _NW - 1,
                      (_NBLK - (_NW - 1) * _WBLK) // (2 * _CB),
                      _WBLK // (2 * _CB))

    pltpu.sync_copy(zeros_hbm, acc_sh.at[pl.ds(s * _SLAB, _SLAB)])
    pltpu.sync_copy(idx_hbm.at[pl.ds(start, _WBLK)], idx_v)
    plsc.subcore_barrier()

    def load(chunk, buf, sem):
        rows0 = (start + chunk * _CB) * _RB
        return pltpu.async_copy(pt_hbm.at[pl.ds(rows0, _CB * _RB)], buf, sem)

    def scat(chunk, buf, sem, j):
        return pltpu.async_copy(buf.at[pl.ds(j * _RB, _RB)],
                                acc_sh.at[idx_v.at[chunk * _CB + j]],
                                sem, add=True)

    def drain_scat(chunk, buf, sem, j):
        pltpu.make_async_copy(buf.at[pl.ds(j * _RB, _RB)],
                              acc_sh.at[idx_v.at[chunk * _CB + j]],
                              sem).wait()

    load(0, buf0_v, sem_l0)

    def body(i, carry):
        pltpu.make_async_copy(pt_hbm.at[pl.ds(0, _CB * _RB)], buf0_v,
                              sem_l0).wait()
        for j in range(_CB):
            scat(2 * i, buf0_v, sem_s0, j)
        @pl.when(i > 0)
        def _():
            for j in range(_CB):
                drain_scat(2 * i - 1, buf1_v, sem_s1, j)
        load(2 * i + 1, buf1_v, sem_l1)
        for j in range(_CB):
            drain_scat(2 * i, buf0_v, sem_s0, j)
        @pl.when(i + 1 < npair)
        def _():
            load(2 * i + 2, buf0_v, sem_l0)
        pltpu.make_async_copy(pt_hbm.at[pl.ds(0, _CB * _RB)], buf1_v,
                              sem_l1).wait()
        for j in range(_CB):
            scat(2 * i + 1, buf1_v, sem_s1, j)
        return carry

    lax.fori_loop(0, npair, body, 0)
    for j in range(_CB):
        drain_scat(2 * npair - 1, buf1_v, sem_s1, j)
    plsc.subcore_barrier()

    pltpu.sync_copy(acc_sh.at[pl.ds(s * _SLAB, _SLAB)],
                    out_hbm.at[c, pl.ds(s * _SLAB, _SLAB)])


def _segment_sum_sc(pt, idx2d, zeros_slab):
    mesh = plsc.VectorSubcoreMesh(core_axis_name="c", subcore_axis_name="s")
    k = pl.kernel(
        _segment_sum_sc_body,
        out_type=jax.ShapeDtypeStruct((_NC, _SP, D), jnp.float32),
        mesh=mesh,
        scratch_types=[
            pltpu.VMEM((_WBLK, _RB), jnp.int32),
            pltpu.VMEM((_CB * _RB, D), jnp.float32),
            pltpu.VMEM((_CB * _RB, D), jnp.float32),
            pltpu.VMEM_SHARED((_SP, D), jnp.float32),
            pltpu.SemaphoreType.DMA,
            pltpu.SemaphoreType.DMA,
            pltpu.SemaphoreType.DMA,
            pltpu.SemaphoreType.DMA,
        ],
    )
    return k(pt, idx2d, zeros_slab)


def kernel(x, idx, W1p, b1p, W2p, b2p, W1r, b1r, W2r, b2r):
    pt = _point_net(x, W1p, b1p, W2p, b2p)
    idx2d = idx.astype(jnp.int32).reshape(_NBLK, _RB)
    idx2d = jnp.pad(idx2d, ((0, _NBLK_PAD - _NBLK), (0, 0)))
    zeros_slab = jnp.zeros((_SLAB, D), jnp.float32)
    parts = _segment_sum_sc(pt, idx2d, zeros_slab)
    return _reduce_net(parts, W1r, b1r, W2r, b2r)
